# Initial kernel scaffold; baseline (speedup 1.0000x reference)
import jax
import jax.numpy as jnp
from jax.experimental import pallas as pl


def kernel(x, edge_index, edge_attr, batch, W1, a_src1, a_dst1, We1, a_edge1, b1, Wel, bel, W2, a_src2, a_dst2, We2, a_edge2, b2, Wout, bout):
    raise NotImplementedError("write your pallas kernel here")


# speedup vs baseline: 38.6266x; 38.6266x over previous
import functools

import jax
import jax.numpy as jnp
from jax import lax
from jax.experimental import pallas as pl
from jax.experimental.pallas import tpu as pltpu
from jax.experimental.pallas import tpu_sc as plsc

N = 10000
E = 320000
DF = 128
DE = 16
U = 32
G = 64

NP = 10240
W = 48
NTILES = 32
CHUNK = 128
NCHUNK = 80
NBUF = 4
EP = NTILES * NCHUNK * CHUNK
NEG = -1e30

f32 = jnp.float32


def _node1_body(x_ref, w1_ref, a1_ref, h_ref, sd_ref):
    h = jnp.dot(x_ref[...], w1_ref[...], preferred_element_type=f32)
    h_ref[:, 0:U] = h
    h_ref[:, U:W] = jnp.zeros((h.shape[0], W - U), f32)
    sd_ref[...] = jnp.dot(h, a1_ref[...], preferred_element_type=f32)


def _node1(x_pad, W1, A1):
    blk = 1024
    grid = NP // blk
    return pl.pallas_call(
        _node1_body,
        grid=(grid,),
        in_specs=[
            pl.BlockSpec((blk, DF), lambda i: (i, 0)),
            pl.BlockSpec((DF, U), lambda i: (0, 0)),
            pl.BlockSpec((U, 2), lambda i: (0, 0)),
        ],
        out_specs=[
            pl.BlockSpec((blk, W), lambda i: (i, 0)),
            pl.BlockSpec((blk, 2), lambda i: (i, 0)),
        ],
        out_shape=[
            jax.ShapeDtypeStruct((NP, W), f32),
            jax.ShapeDtypeStruct((NP, 2), f32),
        ],
    )(x_pad, W1, A1)


TILE_E = EP // NTILES


def _edge_body(eaT_ref, wqT_ref, bqT_ref, q_ref):
    i = pl.program_id(0)
    q = jnp.dot(wqT_ref[...], eaT_ref[...], preferred_element_type=f32)
    q = q + bqT_ref[...]
    cols = i * TILE_E + lax.broadcasted_iota(jnp.int32, (2, TILE_E), 1)
    q = jnp.where(cols < E, q, NEG)
    q_ref[...] = q.reshape(1, 2, NCHUNK, CHUNK)


def _edgeq(eaT, WqT, bqT):
    return pl.pallas_call(
        _edge_body,
        grid=(NTILES,),
        in_specs=[
            pl.BlockSpec((DE, TILE_E), lambda i: (0, i)),
            pl.BlockSpec((2, DE), lambda i: (0, 0)),
            pl.BlockSpec((2, 1), lambda i: (0, 0)),
        ],
        out_specs=pl.BlockSpec((1, 2, NCHUNK, CHUNK), lambda i: (i, 0, 0, 0)),
        out_shape=jax.ShapeDtypeStruct((NTILES, 2, NCHUNK, CHUNK), f32),
    )(eaT, WqT, bqT)


def _mid_body(p_ref, w2_ref, a2_ref, v2_ref, b1_ref, g_ref, sd_ref):
    p = p_ref[...]
    num = p[0, :, 0:U] + p[1, :, 0:U]
    den = p[0, :, U] + p[1, :, U]
    h1 = num / (den + 1e-16)[:, None] + b1_ref[...]
    g = jnp.dot(h1, w2_ref[...], preferred_element_type=f32)
    g_ref[:, 0:U] = g
    g_ref[:, U:W] = jnp.zeros((g.shape[0], W - U), f32)
    sd_ref[...] = (jnp.dot(g, a2_ref[...], preferred_element_type=f32)
                   + jnp.dot(h1, v2_ref[...], preferred_element_type=f32))


def _mid(p1, W2, A2, V2, b1):
    blk = 1024
    grid = NP // blk
    return pl.pallas_call(
        _mid_body,
        grid=(grid,),
        in_specs=[
            pl.BlockSpec((2, blk, W), lambda i: (0, i, 0)),
            pl.BlockSpec((U, U), lambda i: (0, 0)),
            pl.BlockSpec((U, 2), lambda i: (0, 0)),
            pl.BlockSpec((U, 2), lambda i: (0, 0)),
            pl.BlockSpec((1, U), lambda i: (0, 0)),
        ],
        out_specs=[
            pl.BlockSpec((blk, W), lambda i: (i, 0)),
            pl.BlockSpec((blk, 2), lambda i: (i, 0)),
        ],
        out_shape=[
            jax.ShapeDtypeStruct((NP, W), f32),
            jax.ShapeDtypeStruct((NP, 2), f32),
        ],
    )(p1, W2, A2, V2, b1)


def _final_body(p_ref, bf_ref, b2_ref, wout_ref, bout_ref, pred_ref, acc_ref):
    i = pl.program_id(0)
    nsteps = pl.num_programs(0)

    @pl.when(i == 0)
    def _init():
        acc_ref[...] = jnp.zeros_like(acc_ref)

    p = p_ref[...]
    num = p[0, :, 0:U] + p[1, :, 0:U]
    den = p[0, :, U] + p[1, :, U]
    h2 = num / (den + 1e-16)[:, None] + b2_ref[...]
    blk = h2.shape[0]
    aug = jnp.concatenate(
        [h2, jnp.ones((blk, 1), f32), jnp.zeros((blk, W - U - 1), f32)], axis=1)
    gids = lax.broadcasted_iota(jnp.int32, (1, G), 1).astype(f32)
    onehot = (bf_ref[...][:, None] == gids).astype(f32)
    acc_ref[...] += jnp.dot(onehot.T, aug, preferred_element_type=f32)

    @pl.when(i == nsteps - 1)
    def _fin():
        gsum = acc_ref[:, 0:U]
        cnt = acc_ref[:, U]
        gmean = gsum / jnp.maximum(cnt, 1.0)[:, None]
        pred_ref[...] = (jnp.dot(gmean, wout_ref[...],
                                 preferred_element_type=f32) + bout_ref[...])


def _final(p2, batchf, b2, Wout, bout):
    blk = 1024
    grid = NP // blk
    return pl.pallas_call(
        _final_body,
        grid=(grid,),
        in_specs=[
            pl.BlockSpec((2, blk, W), lambda i: (0, i, 0)),
            pl.BlockSpec((blk,), lambda i: (i,)),
            pl.BlockSpec((1, U), lambda i: (0, 0)),
            pl.BlockSpec((U, 1), lambda i: (0, 0)),
            pl.BlockSpec((1, 1), lambda i: (0, 0)),
        ],
        out_specs=pl.BlockSpec((G, 1), lambda i: (0, 0)),
        out_shape=jax.ShapeDtypeStruct((G, 1), f32),
        scratch_shapes=[pltpu.VMEM((G, W), f32)],
    )(p2, batchf, b2, Wout, bout)


def _sc_layer_body(h_hbm, s_hbm, d_hbm, src_hbm, dst_hbm, q_hbm,
                   out_hbm,
                   s_v, d_v, src_v, dst_v, q_v, rin, rout, acc_sp,
                   tsem, gsem, ssem, *, col):
    cid = lax.axis_index("c")
    sid = lax.axis_index("s")
    gtid = cid * 16 + sid

    pltpu.async_copy(s_hbm, s_v, tsem)
    pltpu.async_copy(d_hbm, d_v, tsem)
    pltpu.async_copy(src_hbm.at[gtid], src_v, tsem)
    pltpu.async_copy(dst_hbm.at[gtid], dst_v, tsem)
    pltpu.async_copy(q_hbm.at[gtid, col], q_v, tsem)

    zeros16 = jnp.zeros((16,), f32)

    def _zrow(i, _):
        rin[0][i, pl.ds(0, 16)] = zeros16
        rin[0][i, pl.ds(16, 16)] = zeros16
        rin[0][i, pl.ds(32, 16)] = zeros16
        return 0

    lax.fori_loop(0, CHUNK, _zrow, 0)
    base = sid * 640

    def _zchunk(b, _):
        pltpu.sync_copy(rin[0], acc_sp.at[pl.ds(base + b * CHUNK, CHUNK)])
        return 0

    lax.fori_loop(0, (NP // 16) // CHUNK, _zchunk, 0)

    pltpu.make_async_copy(s_hbm, s_v, tsem).wait()
    pltpu.make_async_copy(d_hbm, d_v, tsem).wait()
    pltpu.make_async_copy(src_hbm.at[gtid], src_v, tsem).wait()
    pltpu.make_async_copy(dst_hbm.at[gtid], dst_v, tsem).wait()
    pltpu.make_async_copy(q_hbm.at[gtid, col], q_v, tsem).wait()
    plsc.subcore_barrier()

    lane0 = lax.broadcasted_iota(jnp.int32, (16,), 0) == 0

    def _gather(j, b):
        pltpu.async_copy(h_hbm.at[src_v.at[j]], rin[b], gsem[b])

    def _wait_gather(j, b):
        pltpu.make_async_copy(h_hbm.at[src_v.at[j]], rin[b], gsem[b]).wait()

    def _scatter(j, b):
        pltpu.async_copy(rout[b], acc_sp.at[dst_v.at[j]], ssem[b], add=True)

    def _wait_scatter(j, b):
        pltpu.make_async_copy(rout[b], acc_sp.at[dst_v.at[j]],
                              ssem[b]).wait()

    def _compute(j, b):
        for k in range(CHUNK // 16):
            sl = pl.ds(16 * k, 16)
            sv = plsc.load_gather(s_v, [src_v[j, sl]])
            dv = plsc.load_gather(d_v, [dst_v[j, sl]])
            e = sv + dv + q_v[j, sl]
            e = jnp.where(e > 0.0, e, 0.2 * e)
            ex = jnp.exp(e)
            for l in range(16):
                i = 16 * k + l
                a = ex[l]
                rout[b][i, pl.ds(0, 16)] = rin[b][i, pl.ds(0, 16)] * a
                rout[b][i, pl.ds(16, 16)] = rin[b][i, pl.ds(16, 16)] * a
                rout[b][i, pl.ds(32, 16)] = jnp.where(lane0, a, 0.0)

    for b in range(NBUF):
        _gather(b, b)

    def _step(g, _):
        for b in range(NBUF):
            j = NBUF * g + b
            _wait_gather(j, b)

            @pl.when(g > 0)
            def _drain():
                _wait_scatter(j - NBUF, b)

            _compute(j, b)
            _scatter(j, b)

            @pl.when(j + NBUF < NCHUNK)
            def _next():
                _gather(j + NBUF, b)
        return 0

    lax.fori_loop(0, NCHUNK // NBUF, _step, 0)
    for b in range(NBUF):
        _wait_scatter(NCHUNK - NBUF + b, b)
    plsc.subcore_barrier()

    pltpu.sync_copy(acc_sp.at[pl.ds(base, 640)],
                    out_hbm.at[cid, pl.ds(base, 640)])


def _sc_layer(h_tab, s_tab, d_tab, src3, dst3, qall, col):
    mesh = plsc.VectorSubcoreMesh(core_axis_name="c", subcore_axis_name="s",
                                  num_cores=2, num_subcores=16)
    return pl.kernel(
        functools.partial(_sc_layer_body, col=col),
        out_type=jax.ShapeDtypeStruct((2, NP, W), f32),
        mesh=mesh,
        compiler_params=pltpu.CompilerParams(needs_layout_passes=False,
                                             use_tc_tiling_on_sc=False),
        scratch_types=[
            pltpu.VMEM((NP,), f32),
            pltpu.VMEM((NP,), f32),
            pltpu.VMEM((NCHUNK, CHUNK), jnp.int32),
            pltpu.VMEM((NCHUNK, CHUNK), jnp.int32),
            pltpu.VMEM((NCHUNK, CHUNK), f32),
            [pltpu.VMEM((CHUNK, W), f32) for _ in range(NBUF)],
            [pltpu.VMEM((CHUNK, W), f32) for _ in range(NBUF)],
            pltpu.VMEM_SHARED((NP, W), f32),
            pltpu.SemaphoreType.DMA,
            [pltpu.SemaphoreType.DMA for _ in range(NBUF)],
            [pltpu.SemaphoreType.DMA for _ in range(NBUF)],
        ],
    )(h_tab, s_tab, d_tab, src3, dst3, qall)


@jax.jit
def kernel(x, edge_index, edge_attr, batch, W1, a_src1, a_dst1, We1, a_edge1,
           b1, Wel, bel, W2, a_src2, a_dst2, We2, a_edge2, b2, Wout, bout):
    A1 = jnp.stack([a_src1, a_dst1], axis=1)
    w2ae = We2 @ a_edge2
    vs = Wel[:U] @ w2ae
    vd = Wel[U:2 * U] @ w2ae
    ve = Wel[2 * U:] @ w2ae
    c0 = bel @ w2ae
    WqT = jnp.stack([We1 @ a_edge1, ve], axis=0)
    bqT = jnp.stack([jnp.zeros((), f32), c0])[:, None]
    A2 = jnp.stack([a_src2, a_dst2], axis=1)
    V2 = jnp.stack([vs, vd], axis=1)

    x_pad = jnp.pad(x, ((0, NP - N), (0, 0)))
    eaT = jnp.pad(edge_attr.T, ((0, 0), (0, EP - E)))
    src3 = jnp.pad(edge_index[0], (0, EP - E)).reshape(NTILES, NCHUNK, CHUNK)
    dst3 = jnp.pad(edge_index[1], (0, EP - E)).reshape(NTILES, NCHUNK, CHUNK)
    batchf = jnp.pad(batch.astype(f32), (0, NP - N), constant_values=float(G))

    h1_tab, sd1 = _node1(x_pad, W1, A1)
    qall = _edgeq(eaT, WqT, bqT)

    p1 = _sc_layer(h1_tab, sd1[:, 0], sd1[:, 1], src3, dst3, qall, 0)

    g_tab, sd2 = _mid(p1, W2, A2, V2, b1[None, :])

    p2 = _sc_layer(g_tab, sd2[:, 0], sd2[:, 1], src3, dst3, qall, 1)

    return _final(p2, batchf, b2[None, :], Wout, bout[None, :])


# speedup vs baseline: 77.2061x; 1.9988x over previous
import functools

import jax
import jax.numpy as jnp
from jax import lax
from jax.experimental import pallas as pl
from jax.experimental.pallas import tpu as pltpu
from jax.experimental.pallas import tpu_sc as plsc

N = 10000
E = 320000
DF = 128
DE = 16
U = 32
G = 64

NP = 10240
W = 48
WH = 32
NTILES = 32
CHUNK = 128
NCHUNK = 80
NBUF = 2
EP = NTILES * NCHUNK * CHUNK
NEG = -1e30

f32 = jnp.float32


def _node1_body(x_ref, w1_ref, a1_ref, h_ref, sd_ref):
    h = jnp.dot(x_ref[...], w1_ref[...], preferred_element_type=f32)
    h_ref[...] = h
    sd_ref[...] = jnp.dot(h, a1_ref[...], preferred_element_type=f32)


def _node1(x_pad, W1, A1):
    blk = 1024
    grid = NP // blk
    return pl.pallas_call(
        _node1_body,
        grid=(grid,),
        in_specs=[
            pl.BlockSpec((blk, DF), lambda i: (i, 0)),
            pl.BlockSpec((DF, U), lambda i: (0, 0)),
            pl.BlockSpec((U, 2), lambda i: (0, 0)),
        ],
        out_specs=[
            pl.BlockSpec((blk, WH), lambda i: (i, 0)),
            pl.BlockSpec((blk, 2), lambda i: (i, 0)),
        ],
        out_shape=[
            jax.ShapeDtypeStruct((NP, WH), f32),
            jax.ShapeDtypeStruct((NP, 2), f32),
        ],
    )(x_pad, W1, A1)


TILE_E = EP // NTILES


def _edge_body(eaT_ref, wqT_ref, bqT_ref, q_ref):
    i = pl.program_id(0)
    q = jnp.dot(wqT_ref[...], eaT_ref[...], preferred_element_type=f32)
    q = q + bqT_ref[...]
    cols = i * TILE_E + lax.broadcasted_iota(jnp.int32, (2, TILE_E), 1)
    q = jnp.where(cols < E, q, NEG)
    q_ref[...] = q.reshape(1, 2, NCHUNK, CHUNK)


def _edgeq(eaT, WqT, bqT):
    return pl.pallas_call(
        _edge_body,
        grid=(NTILES,),
        in_specs=[
            pl.BlockSpec((DE, TILE_E), lambda i: (0, i)),
            pl.BlockSpec((2, DE), lambda i: (0, 0)),
            pl.BlockSpec((2, 1), lambda i: (0, 0)),
        ],
        out_specs=pl.BlockSpec((1, 2, NCHUNK, CHUNK), lambda i: (i, 0, 0, 0)),
        out_shape=jax.ShapeDtypeStruct((NTILES, 2, NCHUNK, CHUNK), f32),
    )(eaT, WqT, bqT)


def _mid_body(p_ref, w2_ref, a2_ref, v2_ref, b1_ref, g_ref, sd_ref):
    p = p_ref[...]
    num = p[0, :, 0:U] + p[1, :, 0:U]
    den = p[0, :, U] + p[1, :, U]
    h1 = num / (den + 1e-16)[:, None] + b1_ref[...]
    g = jnp.dot(h1, w2_ref[...], preferred_element_type=f32)
    g_ref[...] = g
    sd_ref[...] = (jnp.dot(g, a2_ref[...], preferred_element_type=f32)
                   + jnp.dot(h1, v2_ref[...], preferred_element_type=f32))


def _mid(p1, W2, A2, V2, b1):
    blk = 1024
    grid = NP // blk
    return pl.pallas_call(
        _mid_body,
        grid=(grid,),
        in_specs=[
            pl.BlockSpec((2, blk, W), lambda i: (0, i, 0)),
            pl.BlockSpec((U, U), lambda i: (0, 0)),
            pl.BlockSpec((U, 2), lambda i: (0, 0)),
            pl.BlockSpec((U, 2), lambda i: (0, 0)),
            pl.BlockSpec((1, U), lambda i: (0, 0)),
        ],
        out_specs=[
            pl.BlockSpec((blk, WH), lambda i: (i, 0)),
            pl.BlockSpec((blk, 2), lambda i: (i, 0)),
        ],
        out_shape=[
            jax.ShapeDtypeStruct((NP, WH), f32),
            jax.ShapeDtypeStruct((NP, 2), f32),
        ],
    )(p1, W2, A2, V2, b1)


def _final_body(p_ref, bf_ref, b2_ref, wout_ref, bout_ref, pred_ref, acc_ref):
    i = pl.program_id(0)
    nsteps = pl.num_programs(0)

    @pl.when(i == 0)
    def _init():
        acc_ref[...] = jnp.zeros_like(acc_ref)

    p = p_ref[...]
    num = p[0, :, 0:U] + p[1, :, 0:U]
    den = p[0, :, U] + p[1, :, U]
    h2 = num / (den + 1e-16)[:, None] + b2_ref[...]
    blk = h2.shape[0]
    aug = jnp.concatenate(
        [h2, jnp.ones((blk, 1), f32), jnp.zeros((blk, W - U - 1), f32)], axis=1)
    gids = lax.broadcasted_iota(jnp.int32, (1, G), 1).astype(f32)
    onehot = (bf_ref[...][:, None] == gids).astype(f32)
    acc_ref[...] += jnp.dot(onehot.T, aug, preferred_element_type=f32)

    @pl.when(i == nsteps - 1)
    def _fin():
        gsum = acc_ref[:, 0:U]
        cnt = acc_ref[:, U]
        gmean = gsum / jnp.maximum(cnt, 1.0)[:, None]
        pred_ref[...] = (jnp.dot(gmean, wout_ref[...],
                                 preferred_element_type=f32) + bout_ref[...])


def _final(p2, batchf, b2, Wout, bout):
    blk = 1024
    grid = NP // blk
    return pl.pallas_call(
        _final_body,
        grid=(grid,),
        in_specs=[
            pl.BlockSpec((2, blk, W), lambda i: (0, i, 0)),
            pl.BlockSpec((blk,), lambda i: (i,)),
            pl.BlockSpec((1, U), lambda i: (0, 0)),
            pl.BlockSpec((U, 1), lambda i: (0, 0)),
            pl.BlockSpec((1, 1), lambda i: (0, 0)),
        ],
        out_specs=pl.BlockSpec((G, 1), lambda i: (0, 0)),
        out_shape=jax.ShapeDtypeStruct((G, 1), f32),
        scratch_shapes=[pltpu.VMEM((G, W), f32)],
    )(p2, batchf, b2, Wout, bout)


def _sc_layer_body(h_hbm, s_hbm, d_hbm, src_hbm, dst_hbm, q_hbm,
                   out_hbm,
                   s_v, d_v, src_v, dst_v, q_v, rin, rout, acc_sp, h_sp,
                   tsem, gsem, ssem, *, col):
    cid = lax.axis_index("c")
    sid = lax.axis_index("s")
    gtid = cid * 16 + sid

    pltpu.async_copy(s_hbm, s_v, tsem)
    pltpu.async_copy(d_hbm, d_v, tsem)
    pltpu.async_copy(src_hbm.at[gtid], src_v, tsem)
    pltpu.async_copy(dst_hbm.at[gtid], dst_v, tsem)
    pltpu.async_copy(q_hbm.at[gtid, col], q_v, tsem)

    zeros16 = jnp.zeros((16,), f32)

    def _zrow(i, _):
        rout[0][i, pl.ds(0, 16)] = zeros16
        rout[0][i, pl.ds(16, 16)] = zeros16
        rout[0][i, pl.ds(32, 16)] = zeros16
        return 0

    lax.fori_loop(0, CHUNK, _zrow, 0)
    base = sid * 640

    def _zchunk(b, _):
        pltpu.sync_copy(rout[0], acc_sp.at[pl.ds(base + b * CHUNK, CHUNK)])
        return 0

    lax.fori_loop(0, (NP // 16) // CHUNK, _zchunk, 0)

    for b in range(NBUF):
        pltpu.async_copy(h_hbm.at[pl.ds(base + b * CHUNK, CHUNK)],
                         rin[b], gsem[b])
    for v in range(5):
        b = v % NBUF
        pltpu.make_async_copy(h_hbm.at[pl.ds(base + v * CHUNK, CHUNK)],
                              rin[b], gsem[b]).wait()
        pltpu.sync_copy(rin[b], h_sp.at[pl.ds(base + v * CHUNK, CHUNK)])
        if v + NBUF < 5:
            pltpu.async_copy(h_hbm.at[pl.ds(base + (v + NBUF) * CHUNK, CHUNK)],
                             rin[b], gsem[b])

    pltpu.make_async_copy(s_hbm, s_v, tsem).wait()
    pltpu.make_async_copy(d_hbm, d_v, tsem).wait()
    pltpu.make_async_copy(src_hbm.at[gtid], src_v, tsem).wait()
    pltpu.make_async_copy(dst_hbm.at[gtid], dst_v, tsem).wait()
    pltpu.make_async_copy(q_hbm.at[gtid, col], q_v, tsem).wait()
    plsc.subcore_barrier()

    lane0 = lax.broadcasted_iota(jnp.int32, (16,), 0) == 0

    def _gather(j, b):
        pltpu.async_copy(h_sp.at[src_v.at[j]], rin[b], gsem[b])

    def _wait_gather(j, b):
        pltpu.make_async_copy(h_sp.at[src_v.at[j]], rin[b], gsem[b]).wait()

    def _scatter(j, b):
        pltpu.async_copy(rout[b], acc_sp.at[dst_v.at[j]], ssem[b], add=True)

    def _wait_scatter(j, b):
        pltpu.make_async_copy(rout[b], acc_sp.at[dst_v.at[j]],
                              ssem[b]).wait()

    def _compute(j, b):
        for k in range(CHUNK // 16):
            sl = pl.ds(16 * k, 16)
            sv = plsc.load_gather(s_v, [src_v[j, sl]])
            dv = plsc.load_gather(d_v, [dst_v[j, sl]])
            e = sv + dv + q_v[j, sl]
            e = jnp.where(e > 0.0, e, 0.2 * e)
            ex = jnp.exp(e)
            for l in range(16):
                i = 16 * k + l
                a = ex[l]
                rout[b][i, pl.ds(0, 16)] = rin[b][i, pl.ds(0, 16)] * a
                rout[b][i, pl.ds(16, 16)] = rin[b][i, pl.ds(16, 16)] * a
                rout[b][i, pl.ds(32, 16)] = jnp.where(lane0, a, 0.0)

    for b in range(NBUF):
        _gather(b, b)

    def _step(g, _):
        for b in range(NBUF):
            j = NBUF * g + b
            _wait_gather(j, b)

            @pl.when(g > 0)
            def _drain():
                _wait_scatter(j - NBUF, b)

            _compute(j, b)
            _scatter(j, b)

            @pl.when(j + NBUF < NCHUNK)
            def _next():
                _gather(j + NBUF, b)
        return 0

    lax.fori_loop(0, NCHUNK // NBUF, _step, 0)
    for b in range(NBUF):
        _wait_scatter(NCHUNK - NBUF + b, b)
    plsc.subcore_barrier()

    pltpu.sync_copy(acc_sp.at[pl.ds(base, 640)],
                    out_hbm.at[cid, pl.ds(base, 640)])


def _sc_layer(h_tab, s_tab, d_tab, src3, dst3, qall, col):
    mesh = plsc.VectorSubcoreMesh(core_axis_name="c", subcore_axis_name="s",
                                  num_cores=2, num_subcores=16)
    return pl.kernel(
        functools.partial(_sc_layer_body, col=col),
        out_type=jax.ShapeDtypeStruct((2, NP, W), f32),
        mesh=mesh,
        compiler_params=pltpu.CompilerParams(needs_layout_passes=False,
                                             use_tc_tiling_on_sc=False),
        scratch_types=[
            pltpu.VMEM((NP,), f32),
            pltpu.VMEM((NP,), f32),
            pltpu.VMEM((NCHUNK, CHUNK), jnp.int32),
            pltpu.VMEM((NCHUNK, CHUNK), jnp.int32),
            pltpu.VMEM((NCHUNK, CHUNK), f32),
            [pltpu.VMEM((CHUNK, WH), f32) for _ in range(NBUF)],
            [pltpu.VMEM((CHUNK, W), f32) for _ in range(NBUF)],
            pltpu.VMEM_SHARED((NP, W), f32),
            pltpu.VMEM_SHARED((NP, WH), f32),
            pltpu.SemaphoreType.DMA,
            [pltpu.SemaphoreType.DMA for _ in range(NBUF)],
            [pltpu.SemaphoreType.DMA for _ in range(NBUF)],
        ],
    )(h_tab, s_tab, d_tab, src3, dst3, qall)


@jax.jit
def kernel(x, edge_index, edge_attr, batch, W1, a_src1, a_dst1, We1, a_edge1,
           b1, Wel, bel, W2, a_src2, a_dst2, We2, a_edge2, b2, Wout, bout):
    A1 = jnp.stack([a_src1, a_dst1], axis=1)
    w2ae = We2 @ a_edge2
    vs = Wel[:U] @ w2ae
    vd = Wel[U:2 * U] @ w2ae
    ve = Wel[2 * U:] @ w2ae
    c0 = bel @ w2ae
    WqT = jnp.stack([We1 @ a_edge1, ve], axis=0)
    bqT = jnp.stack([jnp.zeros((), f32), c0])[:, None]
    A2 = jnp.stack([a_src2, a_dst2], axis=1)
    V2 = jnp.stack([vs, vd], axis=1)

    x_pad = jnp.pad(x, ((0, NP - N), (0, 0)))
    eaT = jnp.pad(edge_attr.T, ((0, 0), (0, EP - E)))
    src3 = jnp.pad(edge_index[0], (0, EP - E)).reshape(NTILES, NCHUNK, CHUNK)
    dst3 = jnp.pad(edge_index[1], (0, EP - E)).reshape(NTILES, NCHUNK, CHUNK)
    batchf = jnp.pad(batch.astype(f32), (0, NP - N), constant_values=float(G))

    h1_tab, sd1 = _node1(x_pad, W1, A1)
    qall = _edgeq(eaT, WqT, bqT)

    p1 = _sc_layer(h1_tab, sd1[:, 0], sd1[:, 1], src3, dst3, qall, 0)

    g_tab, sd2 = _mid(p1, W2, A2, V2, b1[None, :])

    p2 = _sc_layer(g_tab, sd2[:, 0], sd2[:, 1], src3, dst3, qall, 1)

    return _final(p2, batchf, b2[None, :], Wout, bout[None, :])


# speedup vs baseline: 77.3095x; 1.0013x over previous
import functools

import jax
import jax.numpy as jnp
from jax import lax
from jax.experimental import pallas as pl
from jax.experimental.pallas import tpu as pltpu
from jax.experimental.pallas import tpu_sc as plsc

N = 10000
E = 320000
DF = 128
DE = 16
U = 32
G = 64

NP = 10240
W = 48
WH = 32
NTILES = 32
CHUNK = 128
NCHUNK = 80
NBUF = 2
EP = NTILES * NCHUNK * CHUNK
NEG = -1e30

f32 = jnp.float32


def _node1_body(x_ref, w1_ref, a1_ref, h_ref, sd_ref):
    h = jnp.dot(x_ref[...], w1_ref[...], preferred_element_type=f32)
    h_ref[...] = h
    sd_ref[...] = jnp.dot(h, a1_ref[...], preferred_element_type=f32)


def _node1(x_pad, W1, A1):
    blk = 1024
    grid = NP // blk
    return pl.pallas_call(
        _node1_body,
        grid=(grid,),
        in_specs=[
            pl.BlockSpec((blk, DF), lambda i: (i, 0)),
            pl.BlockSpec((DF, U), lambda i: (0, 0)),
            pl.BlockSpec((U, 2), lambda i: (0, 0)),
        ],
        out_specs=[
            pl.BlockSpec((blk, WH), lambda i: (i, 0)),
            pl.BlockSpec((blk, 2), lambda i: (i, 0)),
        ],
        out_shape=[
            jax.ShapeDtypeStruct((NP, WH), f32),
            jax.ShapeDtypeStruct((NP, 2), f32),
        ],
    )(x_pad, W1, A1)


TILE_E = EP // NTILES


def _edge_body(eaT_ref, wqT_ref, bqT_ref, q_ref):
    i = pl.program_id(0)
    q = jnp.dot(wqT_ref[...], eaT_ref[...], preferred_element_type=f32)
    q = q + bqT_ref[...]
    cols = i * TILE_E + lax.broadcasted_iota(jnp.int32, (2, TILE_E), 1)
    q = jnp.where(cols < E, q, NEG)
    q_ref[...] = q.reshape(1, 2, NCHUNK, CHUNK)


def _edgeq(eaT, WqT, bqT):
    return pl.pallas_call(
        _edge_body,
        grid=(NTILES,),
        in_specs=[
            pl.BlockSpec((DE, TILE_E), lambda i: (0, i)),
            pl.BlockSpec((2, DE), lambda i: (0, 0)),
            pl.BlockSpec((2, 1), lambda i: (0, 0)),
        ],
        out_specs=pl.BlockSpec((1, 2, NCHUNK, CHUNK), lambda i: (i, 0, 0, 0)),
        out_shape=jax.ShapeDtypeStruct((NTILES, 2, NCHUNK, CHUNK), f32),
    )(eaT, WqT, bqT)


def _mid_body(p_ref, w2_ref, a2_ref, v2_ref, b1_ref, g_ref, sd_ref):
    p = p_ref[...]
    num = p[0, :, 0:U] + p[1, :, 0:U]
    den = p[0, :, U] + p[1, :, U]
    h1 = num / (den + 1e-16)[:, None] + b1_ref[...]
    g = jnp.dot(h1, w2_ref[...], preferred_element_type=f32)
    g_ref[...] = g
    sd_ref[...] = (jnp.dot(g, a2_ref[...], preferred_element_type=f32)
                   + jnp.dot(h1, v2_ref[...], preferred_element_type=f32))


def _mid(p1, W2, A2, V2, b1):
    blk = 1024
    grid = NP // blk
    return pl.pallas_call(
        _mid_body,
        grid=(grid,),
        in_specs=[
            pl.BlockSpec((2, blk, W), lambda i: (0, i, 0)),
            pl.BlockSpec((U, U), lambda i: (0, 0)),
            pl.BlockSpec((U, 2), lambda i: (0, 0)),
            pl.BlockSpec((U, 2), lambda i: (0, 0)),
            pl.BlockSpec((1, U), lambda i: (0, 0)),
        ],
        out_specs=[
            pl.BlockSpec((blk, WH), lambda i: (i, 0)),
            pl.BlockSpec((blk, 2), lambda i: (i, 0)),
        ],
        out_shape=[
            jax.ShapeDtypeStruct((NP, WH), f32),
            jax.ShapeDtypeStruct((NP, 2), f32),
        ],
    )(p1, W2, A2, V2, b1)


def _final_body(p_ref, bf_ref, b2_ref, wout_ref, bout_ref, pred_ref, acc_ref):
    i = pl.program_id(0)
    nsteps = pl.num_programs(0)

    @pl.when(i == 0)
    def _init():
        acc_ref[...] = jnp.zeros_like(acc_ref)

    p = p_ref[...]
    num = p[0, :, 0:U] + p[1, :, 0:U]
    den = p[0, :, U] + p[1, :, U]
    h2 = num / (den + 1e-16)[:, None] + b2_ref[...]
    blk = h2.shape[0]
    aug = jnp.concatenate(
        [h2, jnp.ones((blk, 1), f32), jnp.zeros((blk, W - U - 1), f32)], axis=1)
    gids = lax.broadcasted_iota(jnp.int32, (1, G), 1).astype(f32)
    onehot = (bf_ref[...][:, None] == gids).astype(f32)
    acc_ref[...] += jnp.dot(onehot.T, aug, preferred_element_type=f32)

    @pl.when(i == nsteps - 1)
    def _fin():
        gsum = acc_ref[:, 0:U]
        cnt = acc_ref[:, U]
        gmean = gsum / jnp.maximum(cnt, 1.0)[:, None]
        pred_ref[...] = (jnp.dot(gmean, wout_ref[...],
                                 preferred_element_type=f32) + bout_ref[...])


def _final(p2, batchf, b2, Wout, bout):
    blk = 1024
    grid = NP // blk
    return pl.pallas_call(
        _final_body,
        grid=(grid,),
        in_specs=[
            pl.BlockSpec((2, blk, W), lambda i: (0, i, 0)),
            pl.BlockSpec((blk,), lambda i: (i,)),
            pl.BlockSpec((1, U), lambda i: (0, 0)),
            pl.BlockSpec((U, 1), lambda i: (0, 0)),
            pl.BlockSpec((1, 1), lambda i: (0, 0)),
        ],
        out_specs=pl.BlockSpec((G, 1), lambda i: (0, 0)),
        out_shape=jax.ShapeDtypeStruct((G, 1), f32),
        scratch_shapes=[pltpu.VMEM((G, W), f32)],
    )(p2, batchf, b2, Wout, bout)


def _sc_layer_body(h_hbm, s_hbm, d_hbm, src_hbm, dst_hbm, q_hbm,
                   out_hbm,
                   s_v, d_v, src_v, dst_v, q_v, rin, rout, acc_sp, h_sp,
                   tsem, gsem, ssem, *, col):
    cid = lax.axis_index("c")
    sid = lax.axis_index("s")
    gtid = cid * 16 + sid

    pltpu.async_copy(s_hbm, s_v, tsem)
    pltpu.async_copy(d_hbm, d_v, tsem)
    pltpu.async_copy(src_hbm.at[gtid], src_v, tsem)
    pltpu.async_copy(dst_hbm.at[gtid], dst_v, tsem)
    pltpu.async_copy(q_hbm.at[gtid, col], q_v, tsem)

    zeros16 = jnp.zeros((16,), f32)

    def _zrow(i, _):
        for b in range(NBUF):
            rout[b][i, pl.ds(0, 16)] = zeros16
            rout[b][i, pl.ds(16, 16)] = zeros16
            rout[b][i, pl.ds(32, 16)] = zeros16
        return 0

    lax.fori_loop(0, CHUNK, _zrow, 0)
    base = sid * 640

    def _zchunk(b, _):
        pltpu.sync_copy(rout[0], acc_sp.at[pl.ds(base + b * CHUNK, CHUNK)])
        return 0

    lax.fori_loop(0, (NP // 16) // CHUNK, _zchunk, 0)

    for b in range(NBUF):
        pltpu.async_copy(h_hbm.at[pl.ds(base + b * CHUNK, CHUNK)],
                         rin[b], gsem[b])
    for v in range(5):
        b = v % NBUF
        pltpu.make_async_copy(h_hbm.at[pl.ds(base + v * CHUNK, CHUNK)],
                              rin[b], gsem[b]).wait()
        pltpu.sync_copy(rin[b], h_sp.at[pl.ds(base + v * CHUNK, CHUNK)])
        if v + NBUF < 5:
            pltpu.async_copy(h_hbm.at[pl.ds(base + (v + NBUF) * CHUNK, CHUNK)],
                             rin[b], gsem[b])

    pltpu.make_async_copy(s_hbm, s_v, tsem).wait()
    pltpu.make_async_copy(d_hbm, d_v, tsem).wait()
    pltpu.make_async_copy(src_hbm.at[gtid], src_v, tsem).wait()
    pltpu.make_async_copy(dst_hbm.at[gtid], dst_v, tsem).wait()
    pltpu.make_async_copy(q_hbm.at[gtid, col], q_v, tsem).wait()
    plsc.subcore_barrier()

    def _gather(j, b):
        pltpu.async_copy(h_sp.at[src_v.at[j]], rin[b], gsem[b])

    def _wait_gather(j, b):
        pltpu.make_async_copy(h_sp.at[src_v.at[j]], rin[b], gsem[b]).wait()

    def _scatter(j, b):
        pltpu.async_copy(rout[b], acc_sp.at[dst_v.at[j]], ssem[b], add=True)

    def _wait_scatter(j, b):
        pltpu.make_async_copy(rout[b], acc_sp.at[dst_v.at[j]],
                              ssem[b]).wait()

    def _compute(j, b):
        iota16 = lax.broadcasted_iota(jnp.int32, (16,), 0)
        col32 = jnp.full((16,), 32, jnp.int32)
        for k in range(CHUNK // 16):
            sl = pl.ds(16 * k, 16)
            sv = plsc.load_gather(s_v, [src_v[j, sl]])
            dv = plsc.load_gather(d_v, [dst_v[j, sl]])
            e = sv + dv + q_v[j, sl]
            e = jnp.where(e > 0.0, e, 0.2 * e)
            ex = jnp.exp(e)
            for l in range(16):
                i = 16 * k + l
                a = ex[l]
                rout[b][i, pl.ds(0, 16)] = rin[b][i, pl.ds(0, 16)] * a
                rout[b][i, pl.ds(16, 16)] = rin[b][i, pl.ds(16, 16)] * a
            plsc.store_scatter(rout[b], [iota16 + 16 * k, col32], ex)

    for b in range(NBUF):
        _gather(b, b)

    def _step(g, _):
        for b in range(NBUF):
            j = NBUF * g + b
            _wait_gather(j, b)

            @pl.when(g > 0)
            def _drain():
                _wait_scatter(j - NBUF, b)

            _compute(j, b)
            _scatter(j, b)

            @pl.when(j + NBUF < NCHUNK)
            def _next():
                _gather(j + NBUF, b)
        return 0

    lax.fori_loop(0, NCHUNK // NBUF, _step, 0)
    for b in range(NBUF):
        _wait_scatter(NCHUNK - NBUF + b, b)
    plsc.subcore_barrier()

    pltpu.sync_copy(acc_sp.at[pl.ds(base, 640)],
                    out_hbm.at[cid, pl.ds(base, 640)])


def _sc_layer(h_tab, s_tab, d_tab, src3, dst3, qall, col):
    mesh = plsc.VectorSubcoreMesh(core_axis_name="c", subcore_axis_name="s",
                                  num_cores=2, num_subcores=16)
    return pl.kernel(
        functools.partial(_sc_layer_body, col=col),
        out_type=jax.ShapeDtypeStruct((2, NP, W), f32),
        mesh=mesh,
        compiler_params=pltpu.CompilerParams(needs_layout_passes=False,
                                             use_tc_tiling_on_sc=False),
        scratch_types=[
            pltpu.VMEM((NP,), f32),
            pltpu.VMEM((NP,), f32),
            pltpu.VMEM((NCHUNK, CHUNK), jnp.int32),
            pltpu.VMEM((NCHUNK, CHUNK), jnp.int32),
            pltpu.VMEM((NCHUNK, CHUNK), f32),
            [pltpu.VMEM((CHUNK, WH), f32) for _ in range(NBUF)],
            [pltpu.VMEM((CHUNK, W), f32) for _ in range(NBUF)],
            pltpu.VMEM_SHARED((NP, W), f32),
            pltpu.VMEM_SHARED((NP, WH), f32),
            pltpu.SemaphoreType.DMA,
            [pltpu.SemaphoreType.DMA for _ in range(NBUF)],
            [pltpu.SemaphoreType.DMA for _ in range(NBUF)],
        ],
    )(h_tab, s_tab, d_tab, src3, dst3, qall)


@jax.jit
def kernel(x, edge_index, edge_attr, batch, W1, a_src1, a_dst1, We1, a_edge1,
           b1, Wel, bel, W2, a_src2, a_dst2, We2, a_edge2, b2, Wout, bout):
    A1 = jnp.stack([a_src1, a_dst1], axis=1)
    w2ae = We2 @ a_edge2
    vs = Wel[:U] @ w2ae
    vd = Wel[U:2 * U] @ w2ae
    ve = Wel[2 * U:] @ w2ae
    c0 = bel @ w2ae
    WqT = jnp.stack([We1 @ a_edge1, ve], axis=0)
    bqT = jnp.stack([jnp.zeros((), f32), c0])[:, None]
    A2 = jnp.stack([a_src2, a_dst2], axis=1)
    V2 = jnp.stack([vs, vd], axis=1)

    x_pad = jnp.pad(x, ((0, NP - N), (0, 0)))
    eaT = jnp.pad(edge_attr.T, ((0, 0), (0, EP - E)))
    src3 = jnp.pad(edge_index[0], (0, EP - E)).reshape(NTILES, NCHUNK, CHUNK)
    dst3 = jnp.pad(edge_index[1], (0, EP - E)).reshape(NTILES, NCHUNK, CHUNK)
    batchf = jnp.pad(batch.astype(f32), (0, NP - N), constant_values=float(G))

    h1_tab, sd1 = _node1(x_pad, W1, A1)
    qall = _edgeq(eaT, WqT, bqT)

    p1 = _sc_layer(h1_tab, sd1[:, 0], sd1[:, 1], src3, dst3, qall, 0)

    g_tab, sd2 = _mid(p1, W2, A2, V2, b1[None, :])

    p2 = _sc_layer(g_tab, sd2[:, 0], sd2[:, 1], src3, dst3, qall, 1)

    return _final(p2, batchf, b2[None, :], Wout, bout[None, :])


# speedup vs baseline: 80.2258x; 1.0377x over previous
import functools

import jax
import jax.numpy as jnp
from jax import lax
from jax.experimental import pallas as pl
from jax.experimental.pallas import tpu as pltpu
from jax.experimental.pallas import tpu_sc as plsc

N = 10000
E = 320000
DF = 128
DE = 16
U = 32
G = 64

NP = 10240
W = 48
WH = 32
NTILES = 32
CHUNK = 128
NCHUNK = 80
NBUF = 2
EP = NTILES * NCHUNK * CHUNK
NEG = -1e30

f32 = jnp.float32


def _node1_body(x_ref, w1_ref, a1_ref, h_ref, sd_ref):
    h = jnp.dot(x_ref[...], w1_ref[...], preferred_element_type=f32)
    h_ref[...] = h
    sd_ref[...] = lax.dot_general(a1_ref[...], h, (((0,), (1,)), ((), ())),
                                  preferred_element_type=f32)


def _node1(x_pad, W1, A1):
    blk = 1024
    grid = NP // blk
    return pl.pallas_call(
        _node1_body,
        grid=(grid,),
        in_specs=[
            pl.BlockSpec((blk, DF), lambda i: (i, 0)),
            pl.BlockSpec((DF, U), lambda i: (0, 0)),
            pl.BlockSpec((U, 2), lambda i: (0, 0)),
        ],
        out_specs=[
            pl.BlockSpec((blk, WH), lambda i: (i, 0)),
            pl.BlockSpec((2, blk), lambda i: (0, i)),
        ],
        out_shape=[
            jax.ShapeDtypeStruct((NP, WH), f32),
            jax.ShapeDtypeStruct((2, NP), f32),
        ],
    )(x_pad, W1, A1)


TILE_E = EP // NTILES


def _edge_body(eaT_ref, wqT_ref, bqT_ref, q_ref):
    i = pl.program_id(0)
    q = jnp.dot(wqT_ref[...], eaT_ref[...], preferred_element_type=f32)
    q = q + bqT_ref[...]
    cols = i * TILE_E + lax.broadcasted_iota(jnp.int32, (2, TILE_E), 1)
    q = jnp.where(cols < E, q, NEG)
    q_ref[...] = q.reshape(1, 2, NCHUNK, CHUNK)


def _edgeq(eaT, WqT, bqT):
    return pl.pallas_call(
        _edge_body,
        grid=(NTILES,),
        in_specs=[
            pl.BlockSpec((DE, TILE_E), lambda i: (0, i)),
            pl.BlockSpec((2, DE), lambda i: (0, 0)),
            pl.BlockSpec((2, 1), lambda i: (0, 0)),
        ],
        out_specs=pl.BlockSpec((1, 2, NCHUNK, CHUNK), lambda i: (i, 0, 0, 0)),
        out_shape=jax.ShapeDtypeStruct((NTILES, 2, NCHUNK, CHUNK), f32),
    )(eaT, WqT, bqT)


def _mid_body(p_ref, w2_ref, a2_ref, v2_ref, b1_ref, g_ref, sd_ref):
    p = p_ref[...]
    num = p[0, :, 0:U] + p[1, :, 0:U]
    den = p[0, :, U] + p[1, :, U]
    h1 = num / (den + 1e-16)[:, None] + b1_ref[...]
    g = jnp.dot(h1, w2_ref[...], preferred_element_type=f32)
    g_ref[...] = g
    dn = (((0,), (1,)), ((), ()))
    sd_ref[...] = (lax.dot_general(a2_ref[...], g, dn,
                                   preferred_element_type=f32)
                   + lax.dot_general(v2_ref[...], h1, dn,
                                     preferred_element_type=f32))


def _mid(p1, W2, A2, V2, b1):
    blk = 1024
    grid = NP // blk
    return pl.pallas_call(
        _mid_body,
        grid=(grid,),
        in_specs=[
            pl.BlockSpec((2, blk, W), lambda i: (0, i, 0)),
            pl.BlockSpec((U, U), lambda i: (0, 0)),
            pl.BlockSpec((U, 2), lambda i: (0, 0)),
            pl.BlockSpec((U, 2), lambda i: (0, 0)),
            pl.BlockSpec((1, U), lambda i: (0, 0)),
        ],
        out_specs=[
            pl.BlockSpec((blk, WH), lambda i: (i, 0)),
            pl.BlockSpec((2, blk), lambda i: (0, i)),
        ],
        out_shape=[
            jax.ShapeDtypeStruct((NP, WH), f32),
            jax.ShapeDtypeStruct((2, NP), f32),
        ],
    )(p1, W2, A2, V2, b1)


def _final_body(p_ref, bf_ref, b2_ref, wout_ref, bout_ref, pred_ref, acc_ref):
    i = pl.program_id(0)
    nsteps = pl.num_programs(0)

    @pl.when(i == 0)
    def _init():
        acc_ref[...] = jnp.zeros_like(acc_ref)

    p = p_ref[...]
    num = p[0, :, 0:U] + p[1, :, 0:U]
    den = p[0, :, U] + p[1, :, U]
    h2 = num / (den + 1e-16)[:, None] + b2_ref[...]
    blk = h2.shape[0]
    aug = jnp.concatenate(
        [h2, jnp.ones((blk, 1), f32), jnp.zeros((blk, W - U - 1), f32)], axis=1)
    gids = lax.broadcasted_iota(jnp.int32, (1, G), 1).astype(f32)
    onehot = (bf_ref[...][:, None] == gids).astype(f32)
    acc_ref[...] += jnp.dot(onehot.T, aug, preferred_element_type=f32)

    @pl.when(i == nsteps - 1)
    def _fin():
        gsum = acc_ref[:, 0:U]
        cnt = acc_ref[:, U]
        gmean = gsum / jnp.maximum(cnt, 1.0)[:, None]
        pred_ref[...] = (jnp.dot(gmean, wout_ref[...],
                                 preferred_element_type=f32) + bout_ref[...])


def _final(p2, batchf, b2, Wout, bout):
    blk = 1024
    grid = NP // blk
    return pl.pallas_call(
        _final_body,
        grid=(grid,),
        in_specs=[
            pl.BlockSpec((2, blk, W), lambda i: (0, i, 0)),
            pl.BlockSpec((blk,), lambda i: (i,)),
            pl.BlockSpec((1, U), lambda i: (0, 0)),
            pl.BlockSpec((U, 1), lambda i: (0, 0)),
            pl.BlockSpec((1, 1), lambda i: (0, 0)),
        ],
        out_specs=pl.BlockSpec((G, 1), lambda i: (0, 0)),
        out_shape=jax.ShapeDtypeStruct((G, 1), f32),
        scratch_shapes=[pltpu.VMEM((G, W), f32)],
    )(p2, batchf, b2, Wout, bout)


def _sc_layer_body(h_hbm, sd_hbm, src_hbm, dst_hbm, q_hbm,
                   out_hbm,
                   s_v, d_v, src_v, dst_v, q_v, rin, rout, acc_sp, h_sp,
                   tsem, gsem, ssem, *, col):
    cid = lax.axis_index("c")
    sid = lax.axis_index("s")
    gtid = cid * 16 + sid

    pltpu.async_copy(sd_hbm.at[0], s_v, tsem)
    pltpu.async_copy(sd_hbm.at[1], d_v, tsem)
    pltpu.async_copy(src_hbm.at[gtid], src_v, tsem)
    pltpu.async_copy(dst_hbm.at[gtid], dst_v, tsem)
    pltpu.async_copy(q_hbm.at[gtid, col], q_v, tsem)

    zeros16 = jnp.zeros((16,), f32)

    def _zrow(i, _):
        for b in range(NBUF):
            rout[b][i, pl.ds(0, 16)] = zeros16
            rout[b][i, pl.ds(16, 16)] = zeros16
            rout[b][i, pl.ds(32, 16)] = zeros16
        return 0

    lax.fori_loop(0, CHUNK, _zrow, 0)
    base = sid * 640

    def _zchunk(b, _):
        pltpu.sync_copy(rout[0], acc_sp.at[pl.ds(base + b * CHUNK, CHUNK)])
        return 0

    lax.fori_loop(0, (NP // 16) // CHUNK, _zchunk, 0)

    for b in range(NBUF):
        pltpu.async_copy(h_hbm.at[pl.ds(base + b * CHUNK, CHUNK)],
                         rin[b], gsem[b])
    for v in range(5):
        b = v % NBUF
        pltpu.make_async_copy(h_hbm.at[pl.ds(base + v * CHUNK, CHUNK)],
                              rin[b], gsem[b]).wait()
        pltpu.sync_copy(rin[b], h_sp.at[pl.ds(base + v * CHUNK, CHUNK)])
        if v + NBUF < 5:
            pltpu.async_copy(h_hbm.at[pl.ds(base + (v + NBUF) * CHUNK, CHUNK)],
                             rin[b], gsem[b])

    pltpu.make_async_copy(sd_hbm.at[0], s_v, tsem).wait()
    pltpu.make_async_copy(sd_hbm.at[1], d_v, tsem).wait()
    pltpu.make_async_copy(src_hbm.at[gtid], src_v, tsem).wait()
    pltpu.make_async_copy(dst_hbm.at[gtid], dst_v, tsem).wait()
    pltpu.make_async_copy(q_hbm.at[gtid, col], q_v, tsem).wait()
    plsc.subcore_barrier()

    def _gather(j, b):
        pltpu.async_copy(h_sp.at[src_v.at[j]], rin[b], gsem[b])

    def _wait_gather(j, b):
        pltpu.make_async_copy(h_sp.at[src_v.at[j]], rin[b], gsem[b]).wait()

    def _scatter(j, b):
        pltpu.async_copy(rout[b], acc_sp.at[dst_v.at[j]], ssem[b], add=True)

    def _wait_scatter(j, b):
        pltpu.make_async_copy(rout[b], acc_sp.at[dst_v.at[j]],
                              ssem[b]).wait()

    def _compute(j, b):
        iota16 = lax.broadcasted_iota(jnp.int32, (16,), 0)
        col32 = jnp.full((16,), 32, jnp.int32)
        for k in range(CHUNK // 16):
            sl = pl.ds(16 * k, 16)
            sv = plsc.load_gather(s_v, [src_v[j, sl]])
            dv = plsc.load_gather(d_v, [dst_v[j, sl]])
            e = sv + dv + q_v[j, sl]
            e = jnp.where(e > 0.0, e, 0.2 * e)
            ex = jnp.exp(e)
            for l in range(16):
                i = 16 * k + l
                a = ex[l]
                rout[b][i, pl.ds(0, 16)] = rin[b][i, pl.ds(0, 16)] * a
                rout[b][i, pl.ds(16, 16)] = rin[b][i, pl.ds(16, 16)] * a
            plsc.store_scatter(rout[b], [iota16 + 16 * k, col32], ex)

    for b in range(NBUF):
        _gather(b, b)

    def _step(g, _):
        for b in range(NBUF):
            j = NBUF * g + b
            _wait_gather(j, b)

            @pl.when(g > 0)
            def _drain():
                _wait_scatter(j - NBUF, b)

            _compute(j, b)
            _scatter(j, b)

            @pl.when(j + NBUF < NCHUNK)
            def _next():
                _gather(j + NBUF, b)
        return 0

    lax.fori_loop(0, NCHUNK // NBUF, _step, 0)
    for b in range(NBUF):
        _wait_scatter(NCHUNK - NBUF + b, b)
    plsc.subcore_barrier()

    pltpu.sync_copy(acc_sp.at[pl.ds(base, 640)],
                    out_hbm.at[cid, pl.ds(base, 640)])


def _sc_layer(h_tab, sd_tab, src3, dst3, qall, col):
    mesh = plsc.VectorSubcoreMesh(core_axis_name="c", subcore_axis_name="s",
                                  num_cores=2, num_subcores=16)
    return pl.kernel(
        functools.partial(_sc_layer_body, col=col),
        out_type=jax.ShapeDtypeStruct((2, NP, W), f32),
        mesh=mesh,
        compiler_params=pltpu.CompilerParams(needs_layout_passes=False,
                                             use_tc_tiling_on_sc=False),
        scratch_types=[
            pltpu.VMEM((NP,), f32),
            pltpu.VMEM((NP,), f32),
            pltpu.VMEM((NCHUNK, CHUNK), jnp.int32),
            pltpu.VMEM((NCHUNK, CHUNK), jnp.int32),
            pltpu.VMEM((NCHUNK, CHUNK), f32),
            [pltpu.VMEM((CHUNK, WH), f32) for _ in range(NBUF)],
            [pltpu.VMEM((CHUNK, W), f32) for _ in range(NBUF)],
            pltpu.VMEM_SHARED((NP, W), f32),
            pltpu.VMEM_SHARED((NP, WH), f32),
            pltpu.SemaphoreType.DMA,
            [pltpu.SemaphoreType.DMA for _ in range(NBUF)],
            [pltpu.SemaphoreType.DMA for _ in range(NBUF)],
        ],
    )(h_tab, sd_tab, src3, dst3, qall)


@jax.jit
def kernel(x, edge_index, edge_attr, batch, W1, a_src1, a_dst1, We1, a_edge1,
           b1, Wel, bel, W2, a_src2, a_dst2, We2, a_edge2, b2, Wout, bout):
    A1 = jnp.stack([a_src1, a_dst1], axis=1)
    w2ae = We2 @ a_edge2
    vs = Wel[:U] @ w2ae
    vd = Wel[U:2 * U] @ w2ae
    ve = Wel[2 * U:] @ w2ae
    c0 = bel @ w2ae
    WqT = jnp.stack([We1 @ a_edge1, ve], axis=0)
    bqT = jnp.stack([jnp.zeros((), f32), c0])[:, None]
    A2 = jnp.stack([a_src2, a_dst2], axis=1)
    V2 = jnp.stack([vs, vd], axis=1)

    x_pad = jnp.pad(x, ((0, NP - N), (0, 0)))
    eaT = jnp.pad(edge_attr.T, ((0, 0), (0, EP - E)))
    src3 = jnp.pad(edge_index[0], (0, EP - E)).reshape(NTILES, NCHUNK, CHUNK)
    dst3 = jnp.pad(edge_index[1], (0, EP - E)).reshape(NTILES, NCHUNK, CHUNK)
    batchf = jnp.pad(batch.astype(f32), (0, NP - N), constant_values=float(G))

    h1_tab, sd1 = _node1(x_pad, W1, A1)
    qall = _edgeq(eaT, WqT, bqT)

    p1 = _sc_layer(h1_tab, sd1, src3, dst3, qall, 0)

    g_tab, sd2 = _mid(p1, W2, A2, V2, b1[None, :])

    p2 = _sc_layer(g_tab, sd2, src3, dst3, qall, 1)

    return _final(p2, batchf, b2[None, :], Wout, bout[None, :])


# speedup vs baseline: 84.4799x; 1.0530x over previous
import functools

import jax
import jax.numpy as jnp
from jax import lax
from jax.experimental import pallas as pl
from jax.experimental.pallas import tpu as pltpu
from jax.experimental.pallas import tpu_sc as plsc

N = 10000
E = 320000
DF = 128
DE = 16
U = 32
G = 64

NP = 10240
W = 48
WH = 32
NTILES = 32
CHUNK = 128
NCHUNK = 80
NBUF = 2
EP = NTILES * NCHUNK * CHUNK
NEG = -1e30

f32 = jnp.float32


def _node1_body(x_ref, w1_ref, a1_ref, h_ref, sd_ref):
    h = jnp.dot(x_ref[...], w1_ref[...], preferred_element_type=f32)
    h_ref[...] = h
    sd_ref[...] = lax.dot_general(a1_ref[...], h, (((0,), (1,)), ((), ())),
                                  preferred_element_type=f32)


def _node1(x_pad, W1, A1):
    blk = 1024
    grid = NP // blk
    return pl.pallas_call(
        _node1_body,
        grid=(grid,),
        in_specs=[
            pl.BlockSpec((blk, DF), lambda i: (i, 0)),
            pl.BlockSpec((DF, U), lambda i: (0, 0)),
            pl.BlockSpec((U, 2), lambda i: (0, 0)),
        ],
        out_specs=[
            pl.BlockSpec((blk, WH), lambda i: (i, 0)),
            pl.BlockSpec((2, blk), lambda i: (0, i)),
        ],
        out_shape=[
            jax.ShapeDtypeStruct((NP, WH), f32),
            jax.ShapeDtypeStruct((2, NP), f32),
        ],
    )(x_pad, W1, A1)


TILE_E = EP // NTILES


def _edge_body(eaT_ref, wqT_ref, bqT_ref, q_ref):
    i = pl.program_id(0)
    q = jnp.dot(wqT_ref[...], eaT_ref[...], preferred_element_type=f32)
    q = q + bqT_ref[...]
    cols = i * TILE_E + lax.broadcasted_iota(jnp.int32, (2, TILE_E), 1)
    q = jnp.where(cols < E, q, NEG)
    q_ref[...] = q.reshape(1, 2, NCHUNK, CHUNK)


def _edgeq(eaT, WqT, bqT):
    return pl.pallas_call(
        _edge_body,
        grid=(NTILES,),
        in_specs=[
            pl.BlockSpec((DE, TILE_E), lambda i: (0, i)),
            pl.BlockSpec((2, DE), lambda i: (0, 0)),
            pl.BlockSpec((2, 1), lambda i: (0, 0)),
        ],
        out_specs=pl.BlockSpec((1, 2, NCHUNK, CHUNK), lambda i: (i, 0, 0, 0)),
        out_shape=jax.ShapeDtypeStruct((NTILES, 2, NCHUNK, CHUNK), f32),
    )(eaT, WqT, bqT)


def _mid_body(p_ref, w2_ref, a2_ref, v2_ref, b1_ref, g_ref, sd_ref):
    p = p_ref[...]
    num = p[0, :, 0:U] + p[1, :, 0:U]
    den = p[0, :, U] + p[1, :, U]
    h1 = num / (den + 1e-16)[:, None] + b1_ref[...]
    g = jnp.dot(h1, w2_ref[...], preferred_element_type=f32)
    g_ref[...] = g
    dn = (((0,), (1,)), ((), ()))
    sd_ref[...] = (lax.dot_general(a2_ref[...], g, dn,
                                   preferred_element_type=f32)
                   + lax.dot_general(v2_ref[...], h1, dn,
                                     preferred_element_type=f32))


def _mid(p1, W2, A2, V2, b1):
    blk = 1024
    grid = NP // blk
    return pl.pallas_call(
        _mid_body,
        grid=(grid,),
        in_specs=[
            pl.BlockSpec((2, blk, W), lambda i: (0, i, 0)),
            pl.BlockSpec((U, U), lambda i: (0, 0)),
            pl.BlockSpec((U, 2), lambda i: (0, 0)),
            pl.BlockSpec((U, 2), lambda i: (0, 0)),
            pl.BlockSpec((1, U), lambda i: (0, 0)),
        ],
        out_specs=[
            pl.BlockSpec((blk, WH), lambda i: (i, 0)),
            pl.BlockSpec((2, blk), lambda i: (0, i)),
        ],
        out_shape=[
            jax.ShapeDtypeStruct((NP, WH), f32),
            jax.ShapeDtypeStruct((2, NP), f32),
        ],
    )(p1, W2, A2, V2, b1)


def _final_body(p_ref, bf_ref, b2_ref, wout_ref, bout_ref, pred_ref, acc_ref):
    i = pl.program_id(0)
    nsteps = pl.num_programs(0)

    @pl.when(i == 0)
    def _init():
        acc_ref[...] = jnp.zeros_like(acc_ref)

    p = p_ref[...]
    num = p[0, :, 0:U] + p[1, :, 0:U]
    den = p[0, :, U] + p[1, :, U]
    h2 = num / (den + 1e-16)[:, None] + b2_ref[...]
    blk = h2.shape[0]
    aug = jnp.concatenate(
        [h2, jnp.ones((blk, 1), f32), jnp.zeros((blk, W - U - 1), f32)], axis=1)
    gids = lax.broadcasted_iota(jnp.int32, (1, G), 1).astype(f32)
    onehot = (bf_ref[...][:, None] == gids).astype(f32)
    acc_ref[...] += jnp.dot(onehot.T, aug, preferred_element_type=f32)

    @pl.when(i == nsteps - 1)
    def _fin():
        gsum = acc_ref[:, 0:U]
        cnt = acc_ref[:, U]
        gmean = gsum / jnp.maximum(cnt, 1.0)[:, None]
        pred_ref[...] = (jnp.dot(gmean, wout_ref[...],
                                 preferred_element_type=f32) + bout_ref[...])


def _final(p2, batchf, b2, Wout, bout):
    blk = 1024
    grid = NP // blk
    return pl.pallas_call(
        _final_body,
        grid=(grid,),
        in_specs=[
            pl.BlockSpec((2, blk, W), lambda i: (0, i, 0)),
            pl.BlockSpec((blk,), lambda i: (i,)),
            pl.BlockSpec((1, U), lambda i: (0, 0)),
            pl.BlockSpec((U, 1), lambda i: (0, 0)),
            pl.BlockSpec((1, 1), lambda i: (0, 0)),
        ],
        out_specs=pl.BlockSpec((G, 1), lambda i: (0, 0)),
        out_shape=jax.ShapeDtypeStruct((G, 1), f32),
        scratch_shapes=[pltpu.VMEM((G, W), f32)],
    )(p2, batchf, b2, Wout, bout)


def _sc_layer_body(h_hbm, sd_hbm, ei_hbm, q_hbm,
                   out_hbm,
                   s_v, d_v, src_v, dst_v, q_v, rin, rout, acc_sp, h_sp,
                   tsem, gsem, ssem, *, col):
    cid = lax.axis_index("c")
    sid = lax.axis_index("s")
    gtid = cid * 16 + sid

    pltpu.async_copy(sd_hbm.at[0], s_v, tsem)
    pltpu.async_copy(sd_hbm.at[1], d_v, tsem)
    pltpu.async_copy(ei_hbm.at[0, gtid], src_v, tsem)
    pltpu.async_copy(ei_hbm.at[1, gtid], dst_v, tsem)
    pltpu.async_copy(q_hbm.at[gtid, col], q_v, tsem)

    zeros16 = jnp.zeros((16,), f32)

    def _zrow(i, _):
        for b in range(NBUF):
            rout[b][i, pl.ds(0, 16)] = zeros16
            rout[b][i, pl.ds(16, 16)] = zeros16
            rout[b][i, pl.ds(32, 16)] = zeros16
        return 0

    lax.fori_loop(0, CHUNK, _zrow, 0)
    base = sid * 640

    def _zchunk(b, _):
        pltpu.sync_copy(rout[0], acc_sp.at[pl.ds(base + b * CHUNK, CHUNK)])
        return 0

    lax.fori_loop(0, (NP // 16) // CHUNK, _zchunk, 0)

    for b in range(NBUF):
        pltpu.async_copy(h_hbm.at[pl.ds(base + b * CHUNK, CHUNK)],
                         rin[b], gsem[b])
    for v in range(5):
        b = v % NBUF
        pltpu.make_async_copy(h_hbm.at[pl.ds(base + v * CHUNK, CHUNK)],
                              rin[b], gsem[b]).wait()
        pltpu.sync_copy(rin[b], h_sp.at[pl.ds(base + v * CHUNK, CHUNK)])
        if v + NBUF < 5:
            pltpu.async_copy(h_hbm.at[pl.ds(base + (v + NBUF) * CHUNK, CHUNK)],
                             rin[b], gsem[b])

    pltpu.make_async_copy(sd_hbm.at[0], s_v, tsem).wait()
    pltpu.make_async_copy(sd_hbm.at[1], d_v, tsem).wait()
    pltpu.make_async_copy(ei_hbm.at[0, gtid], src_v, tsem).wait()
    pltpu.make_async_copy(ei_hbm.at[1, gtid], dst_v, tsem).wait()
    pltpu.make_async_copy(q_hbm.at[gtid, col], q_v, tsem).wait()
    plsc.subcore_barrier()

    def _gather(j, b):
        pltpu.async_copy(h_sp.at[src_v.at[j]], rin[b], gsem[b])

    def _wait_gather(j, b):
        pltpu.make_async_copy(h_sp.at[src_v.at[j]], rin[b], gsem[b]).wait()

    def _scatter(j, b):
        pltpu.async_copy(rout[b], acc_sp.at[dst_v.at[j]], ssem[b], add=True)

    def _wait_scatter(j, b):
        pltpu.make_async_copy(rout[b], acc_sp.at[dst_v.at[j]],
                              ssem[b]).wait()

    def _compute(j, b):
        iota16 = lax.broadcasted_iota(jnp.int32, (16,), 0)
        col32 = jnp.full((16,), 32, jnp.int32)
        for k in range(CHUNK // 16):
            sl = pl.ds(16 * k, 16)
            sv = plsc.load_gather(s_v, [src_v[j, sl]])
            dv = plsc.load_gather(d_v, [dst_v[j, sl]])
            e = sv + dv + q_v[j, sl]
            e = jnp.where(e > 0.0, e, 0.2 * e)
            ex = jnp.exp(e)
            for l in range(16):
                i = 16 * k + l
                a = ex[l]
                rout[b][i, pl.ds(0, 16)] = rin[b][i, pl.ds(0, 16)] * a
                rout[b][i, pl.ds(16, 16)] = rin[b][i, pl.ds(16, 16)] * a
            plsc.store_scatter(rout[b], [iota16 + 16 * k, col32], ex)

    for b in range(NBUF):
        _gather(b, b)

    def _step(g, _):
        for b in range(NBUF):
            j = NBUF * g + b
            _wait_gather(j, b)

            @pl.when(g > 0)
            def _drain():
                _wait_scatter(j - NBUF, b)

            _compute(j, b)
            _scatter(j, b)

            @pl.when(j + NBUF < NCHUNK)
            def _next():
                _gather(j + NBUF, b)
        return 0

    lax.fori_loop(0, NCHUNK // NBUF, _step, 0)
    for b in range(NBUF):
        _wait_scatter(NCHUNK - NBUF + b, b)
    plsc.subcore_barrier()

    pltpu.sync_copy(acc_sp.at[pl.ds(base, 640)],
                    out_hbm.at[cid, pl.ds(base, 640)])


def _sc_layer(h_tab, sd_tab, ei4, qall, col):
    mesh = plsc.VectorSubcoreMesh(core_axis_name="c", subcore_axis_name="s",
                                  num_cores=2, num_subcores=16)
    return pl.kernel(
        functools.partial(_sc_layer_body, col=col),
        out_type=jax.ShapeDtypeStruct((2, NP, W), f32),
        mesh=mesh,
        compiler_params=pltpu.CompilerParams(needs_layout_passes=False,
                                             use_tc_tiling_on_sc=False),
        scratch_types=[
            pltpu.VMEM((NP,), f32),
            pltpu.VMEM((NP,), f32),
            pltpu.VMEM((NCHUNK, CHUNK), jnp.int32),
            pltpu.VMEM((NCHUNK, CHUNK), jnp.int32),
            pltpu.VMEM((NCHUNK, CHUNK), f32),
            [pltpu.VMEM((CHUNK, WH), f32) for _ in range(NBUF)],
            [pltpu.VMEM((CHUNK, W), f32) for _ in range(NBUF)],
            pltpu.VMEM_SHARED((NP, W), f32),
            pltpu.VMEM_SHARED((NP, WH), f32),
            pltpu.SemaphoreType.DMA,
            [pltpu.SemaphoreType.DMA for _ in range(NBUF)],
            [pltpu.SemaphoreType.DMA for _ in range(NBUF)],
        ],
    )(h_tab, sd_tab, ei4, qall)


@jax.jit
def kernel(x, edge_index, edge_attr, batch, W1, a_src1, a_dst1, We1, a_edge1,
           b1, Wel, bel, W2, a_src2, a_dst2, We2, a_edge2, b2, Wout, bout):
    A1 = jnp.stack([a_src1, a_dst1], axis=1)
    w2ae = We2 @ a_edge2
    vs = Wel[:U] @ w2ae
    vd = Wel[U:2 * U] @ w2ae
    ve = Wel[2 * U:] @ w2ae
    c0 = bel @ w2ae
    WqT = jnp.stack([We1 @ a_edge1, ve], axis=0)
    bqT = jnp.stack([jnp.zeros((), f32), c0])[:, None]
    A2 = jnp.stack([a_src2, a_dst2], axis=1)
    V2 = jnp.stack([vs, vd], axis=1)

    x_pad = jnp.pad(x, ((0, NP - N), (0, 0)))
    eaT = jnp.pad(edge_attr.T, ((0, 0), (0, EP - E)))
    ei4 = jnp.pad(edge_index, ((0, 0), (0, EP - E))).reshape(
        2, NTILES, NCHUNK, CHUNK)
    batchf = jnp.pad(batch.astype(f32), (0, NP - N), constant_values=float(G))

    h1_tab, sd1 = _node1(x_pad, W1, A1)
    qall = _edgeq(eaT, WqT, bqT)

    p1 = _sc_layer(h1_tab, sd1, ei4, qall, 0)

    g_tab, sd2 = _mid(p1, W2, A2, V2, b1[None, :])

    p2 = _sc_layer(g_tab, sd2, ei4, qall, 1)

    return _final(p2, batchf, b2[None, :], Wout, bout[None, :])
